# baseline (speedup 1.0000x reference)
import jax
import jax.numpy as jnp
import numpy as np
from jax.experimental import pallas as pl
from jax.experimental.pallas import tpu as pltpu

_B, _NP, _NG, _TIME = 256, 128, 128, 10
_BB = 128

_DN_TT = (((0,), (0,)), ((), ()))
_DN_NN = (((1,), (0,)), ((), ()))


def _smooth_l1(d):
    a = jnp.abs(d)
    return jnp.where(a < 1.0, 0.5 * a * a, a - 0.5)


def _one_instance(ipxr, ipyr, ppxr, ppyr, gxr, gyr, kpmr, ipxc, ipyc):
    f32 = jnp.float32
    ones = jnp.ones((1, _NP), f32)
    zeros = jnp.zeros((1, _NP), f32)
    gxpr = jnp.concatenate([gxr[:, -1:], gxr[:, :-1]], axis=1)
    gypr = jnp.concatenate([gyr[:, -1:], gyr[:, :-1]], axis=1)

    pn = ipxr * ipxr + ipyr * ipyr
    gg = gxr * gxr + gyr * gyr
    gg2 = gxpr * gxr + gypr * gyr
    hxr = 0.5 * (gxpr + gxr)
    hyr = 0.5 * (gypr + gyr)
    lhs = jnp.concatenate([
        jnp.concatenate([gxr, hxr], axis=1),
        jnp.concatenate([gyr, hyr], axis=1),
        jnp.concatenate([gg, gg2], axis=1),
        jnp.concatenate([ones, ones], axis=1),
    ], axis=0)
    rhs = jnp.concatenate([-2.0 * ipxr, -2.0 * ipyr, ones, pn],
                          axis=0)
    dd = jax.lax.dot_general(lhs, rhs, _DN_TT,
                             preferred_element_type=f32)
    B2 = dd[:_NG]
    C2 = dd[_NG:]
    A2 = jnp.concatenate([B2[-1:], B2[:-1]], axis=0)

    den = (A2 + B2) - 2.0 * C2
    num = A2 - C2
    num2 = num + num
    cstar = jnp.where(den > 0.0, num * pl.reciprocal(den, approx=False), 0.0)
    bc = jnp.clip(jnp.floor(cstar * 10.0 + 0.5), 0.0, 9.0) / 10.0
    bd = A2 - bc * num2 + (bc * bc) * den

    dmin = jnp.min(bd, axis=0, keepdims=True)
    oh = (bd == dmin).astype(f32)
    csel = jnp.sum(oh * bc, axis=0, keepdims=True)
    gx4 = jnp.concatenate([gxr, gxpr, gyr, gypr], axis=0)
    sel4 = jax.lax.dot_general(gx4, oh, _DN_NN,
                               preferred_element_type=f32)
    omc = 1.0 - csel
    nx = sel4[0:1] * csel + sel4[1:2] * omc
    ny = sel4[2:3] * csel + sel4[3:4] * omc
    r1 = _smooth_l1(ppxr - nx) + _smooth_l1(ppyr - ny)

    dxT = ipxc - gxr
    dyT = ipyc - gyr
    B2T = dxT * dxT + dyT * dyT
    dminT = jnp.min(B2T, axis=0, keepdims=True)
    oh2 = (B2T == dminT).astype(f32)
    pp2 = jnp.concatenate([ppxr, ppyr], axis=0)
    sp = jax.lax.dot_general(pp2, oh2, _DN_NN,
                             preferred_element_type=f32)
    l2 = _smooth_l1(sp[0:1] - gxr) + _smooth_l1(sp[1:2] - gyr)
    r2 = l2 * kpmr
    return r1, r2


def _dm_kernel(ipx, ipy, ppx, ppy, gxv, gyv, kpmv, ipxc, ipyc,
               out, s1a, s2a, s3a):
    b = pl.program_id(0)

    @pl.when(b == 0)
    def _init():
        s1a[...] = jnp.zeros_like(s1a)
        s2a[...] = jnp.zeros_like(s2a)
        s3a[...] = jnp.zeros_like(s3a)

    s1 = jnp.zeros((1, _NP), jnp.float32)
    s2 = jnp.zeros((1, _NG), jnp.float32)
    s3 = jnp.zeros((1, _NG), jnp.float32)
    for i in range(_BB):
        r1, r2 = _one_instance(ipx[i], ipy[i], ppx[i], ppy[i],
                               gxv[i], gyv[i], kpmv[i],
                               ipxc[i], ipyc[i])
        s1 = s1 + r1
        s2 = s2 + r2
        s3 = s3 + kpmv[i]

    s1a[...] = s1a[...] + s1
    s2a[...] = s2a[...] + s2
    s3a[...] = s3a[...] + s3

    @pl.when(b == (_B // _BB) - 1)
    def _final():
        t1 = jnp.sum(s1a[...])
        t2 = jnp.sum(s2a[...])
        t3 = jnp.sum(s3a[...])
        loss = 0.5 * (t2 / (2.0 * t3 + 1.0)
                      + t1 / np.float32(_B * _NP * 2))
        out[...] = jnp.broadcast_to(loss, (1, 1))


def _run(ipx3, ipy3, ppx3, ppy3, gx3, gy3, kpm3, ipxc3, ipyc3,
         interpret=False):
    row_spec = pl.BlockSpec((_BB, 1, _NP), lambda b: (b, 0, 0))
    col_spec = pl.BlockSpec((_BB, _NP, 1), lambda b: (b, 0, 0))
    return pl.pallas_call(
        _dm_kernel,
        grid=(_B // _BB,),
        in_specs=[row_spec] * 7 + [col_spec] * 2,
        out_specs=pl.BlockSpec((1, 1), lambda b: (0, 0)),
        out_shape=jax.ShapeDtypeStruct((1, 1), jnp.float32),
        scratch_shapes=[pltpu.VMEM((1, _NP), jnp.float32)] * 3,
        interpret=interpret,
    )(ipx3, ipy3, ppx3, ppy3, gx3, gy3, kpm3, ipxc3, ipyc3)


def kernel(ini_pred_poly, pred_poly, gt_poly, keyPointsMask):
    ipx3 = ini_pred_poly[:, :, 0].reshape(_B, 1, _NP)
    ipy3 = ini_pred_poly[:, :, 1].reshape(_B, 1, _NP)
    ppx3 = pred_poly[:, :, 0].reshape(_B, 1, _NP)
    ppy3 = pred_poly[:, :, 1].reshape(_B, 1, _NP)
    gx3 = gt_poly[:, :, 0].reshape(_B, 1, _NG)
    gy3 = gt_poly[:, :, 1].reshape(_B, 1, _NG)
    kpm3 = keyPointsMask.reshape(_B, 1, _NG)
    ipxc3 = ini_pred_poly[:, :, 0].reshape(_B, _NP, 1)
    ipyc3 = ini_pred_poly[:, :, 1].reshape(_B, _NP, 1)
    out = _run(ipx3, ipy3, ppx3, ppy3, gx3, gy3, kpm3, ipxc3, ipyc3)
    return out[0, 0]


# speedup vs baseline: 1.0509x; 1.0509x over previous
import jax
import jax.numpy as jnp
import numpy as np
from jax.experimental import pallas as pl
from jax.experimental.pallas import tpu as pltpu

_B, _NP, _NG, _TIME = 256, 128, 128, 10
_BB = 64

_DN_TT = (((0,), (0,)), ((), ()))
_DN_NN = (((1,), (0,)), ((), ()))


def _smooth_l1(d):
    a = jnp.abs(d)
    return jnp.where(a < 1.0, 0.5 * a * a, a - 0.5)


def _one_instance(ipxr, ipyr, ppxr, ppyr, gxr, gyr, kpmr, ipxc, ipyc):
    f32 = jnp.float32
    ones = jnp.ones((1, _NP), f32)
    zeros = jnp.zeros((1, _NP), f32)
    gxpr = jnp.concatenate([gxr[:, -1:], gxr[:, :-1]], axis=1)
    gypr = jnp.concatenate([gyr[:, -1:], gyr[:, :-1]], axis=1)

    pn = ipxr * ipxr + ipyr * ipyr
    gg = gxr * gxr + gyr * gyr
    gg2 = gxpr * gxr + gypr * gyr
    hxr = 0.5 * (gxpr + gxr)
    hyr = 0.5 * (gypr + gyr)
    lhs = jnp.concatenate([
        jnp.concatenate([gxr, hxr], axis=1),
        jnp.concatenate([gyr, hyr], axis=1),
        jnp.concatenate([gg, gg2], axis=1),
        jnp.concatenate([ones, ones], axis=1),
    ], axis=0)
    rhs = jnp.concatenate([-2.0 * ipxr, -2.0 * ipyr, ones, pn],
                          axis=0)
    dd = jax.lax.dot_general(lhs, rhs, _DN_TT,
                             preferred_element_type=f32)
    B2 = dd[:_NG]
    C2 = dd[_NG:]
    A2 = jnp.concatenate([B2[-1:], B2[:-1]], axis=0)

    den = (A2 + B2) - 2.0 * C2
    num = A2 - C2
    num2 = num + num
    cstar = jnp.where(den > 0.0, num * pl.reciprocal(den, approx=False), 0.0)
    bc = jnp.clip(jnp.floor(cstar * 10.0 + 0.5), 0.0, 9.0) / 10.0
    bd = A2 - bc * num2 + (bc * bc) * den

    dmin = jnp.min(bd, axis=0, keepdims=True)
    oh = (bd == dmin).astype(f32)
    csel = jnp.sum(oh * bc, axis=0, keepdims=True)
    gx4 = jnp.concatenate([gxr, gxpr, gyr, gypr], axis=0)
    sel4 = jax.lax.dot_general(gx4, oh, _DN_NN,
                               preferred_element_type=f32)
    omc = 1.0 - csel
    nx = sel4[0:1] * csel + sel4[1:2] * omc
    ny = sel4[2:3] * csel + sel4[3:4] * omc
    r1 = _smooth_l1(ppxr - nx) + _smooth_l1(ppyr - ny)

    dxT = ipxc - gxr
    dyT = ipyc - gyr
    B2T = dxT * dxT + dyT * dyT
    dminT = jnp.min(B2T, axis=0, keepdims=True)
    oh2 = (B2T == dminT).astype(f32)
    pp2 = jnp.concatenate([ppxr, ppyr], axis=0)
    sp = jax.lax.dot_general(pp2, oh2, _DN_NN,
                             preferred_element_type=f32)
    l2 = _smooth_l1(sp[0:1] - gxr) + _smooth_l1(sp[1:2] - gyr)
    r2 = l2 * kpmr
    return r1, r2


def _dm_kernel(ipx, ipy, ppx, ppy, gxv, gyv, kpmv, ipxc, ipyc,
               out, s1a, s2a, s3a):
    b = pl.program_id(0)

    @pl.when(b == 0)
    def _init():
        s1a[...] = jnp.zeros_like(s1a)
        s2a[...] = jnp.zeros_like(s2a)
        s3a[...] = jnp.zeros_like(s3a)

    s1 = jnp.zeros((1, _NP), jnp.float32)
    s2 = jnp.zeros((1, _NG), jnp.float32)
    s3 = jnp.zeros((1, _NG), jnp.float32)
    for i in range(_BB):
        r1, r2 = _one_instance(ipx[i], ipy[i], ppx[i], ppy[i],
                               gxv[i], gyv[i], kpmv[i],
                               ipxc[i], ipyc[i])
        s1 = s1 + r1
        s2 = s2 + r2
        s3 = s3 + kpmv[i]

    s1a[...] = s1a[...] + s1
    s2a[...] = s2a[...] + s2
    s3a[...] = s3a[...] + s3

    @pl.when(b == (_B // _BB) - 1)
    def _final():
        t1 = jnp.sum(s1a[...])
        t2 = jnp.sum(s2a[...])
        t3 = jnp.sum(s3a[...])
        loss = 0.5 * (t2 / (2.0 * t3 + 1.0)
                      + t1 / np.float32(_B * _NP * 2))
        out[...] = jnp.broadcast_to(loss, (1, 1))


def _run(ipx3, ipy3, ppx3, ppy3, gx3, gy3, kpm3, ipxc3, ipyc3,
         interpret=False):
    row_spec = pl.BlockSpec((_BB, 1, _NP), lambda b: (b, 0, 0))
    col_spec = pl.BlockSpec((_BB, _NP, 1), lambda b: (b, 0, 0))
    return pl.pallas_call(
        _dm_kernel,
        grid=(_B // _BB,),
        in_specs=[row_spec] * 7 + [col_spec] * 2,
        out_specs=pl.BlockSpec((1, 1), lambda b: (0, 0)),
        out_shape=jax.ShapeDtypeStruct((1, 1), jnp.float32),
        scratch_shapes=[pltpu.VMEM((1, _NP), jnp.float32)] * 3,
        interpret=interpret,
    )(ipx3, ipy3, ppx3, ppy3, gx3, gy3, kpm3, ipxc3, ipyc3)


def kernel(ini_pred_poly, pred_poly, gt_poly, keyPointsMask):
    ipx3 = ini_pred_poly[:, :, 0].reshape(_B, 1, _NP)
    ipy3 = ini_pred_poly[:, :, 1].reshape(_B, 1, _NP)
    ppx3 = pred_poly[:, :, 0].reshape(_B, 1, _NP)
    ppy3 = pred_poly[:, :, 1].reshape(_B, 1, _NP)
    gx3 = gt_poly[:, :, 0].reshape(_B, 1, _NG)
    gy3 = gt_poly[:, :, 1].reshape(_B, 1, _NG)
    kpm3 = keyPointsMask.reshape(_B, 1, _NG)
    ipxc3 = ini_pred_poly[:, :, 0].reshape(_B, _NP, 1)
    ipyc3 = ini_pred_poly[:, :, 1].reshape(_B, _NP, 1)
    out = _run(ipx3, ipy3, ppx3, ppy3, gx3, gy3, kpm3, ipxc3, ipyc3)
    return out[0, 0]
